# baseline (speedup 1.0000x reference)
import jax
import jax.numpy as jnp
from jax import lax
from jax.experimental import pallas as pl
from jax.experimental.pallas import tpu as pltpu
from jax.experimental.pallas import tpu_sc as plsc

B = 1024
L = 20
C = 1000
FLAT = B * C * L
NC, NS = 2, 16
NW = NC * NS
TPB = 5
CPB = 8 * TPB
BLK = TPB * 8 * 128
NBLK = FLAT // BLK
BPL = C // CPB
SLOTS = -(-NBLK // NW)


def _sc_body(t_hbm, z_hbm, out_hbm, t_v, slab0, slab1, sem0, sem1, sem_z):
    core = lax.axis_index("c")
    sub = lax.axis_index("s")
    wid = core * NS + sub

    z0 = pltpu.async_copy(z_hbm, slab0, sem_z)
    z1 = pltpu.async_copy(z_hbm, slab1, sem_z)
    pltpu.sync_copy(t_hbm, t_v)
    z0.wait()
    z1.wait()

    lane = lax.iota(jnp.int32, 16)
    ones16 = jnp.ones((16,), jnp.float32)
    zeros16 = jnp.zeros((16,), jnp.float32)
    slabs = (slab0, slab1)
    sems = (sem0, sem1)

    def inject(slab, blk, val):
        l = blk // BPL
        ct0 = (blk % BPL) * TPB
        c0 = ct0 * 8

        @pl.loop(0, B // 16, unroll=1)
        def body(g):
            b_vec = g * 16 + lane
            cv = plsc.load_gather(t_v, [b_vec * L + l])
            m = (cv >= c0) & (cv < c0 + CPB)
            loc = (
                ((cv >> 3) - ct0) * 8192
                + (b_vec >> 7) * 1024
                + (cv & 7) * 128
                + (b_vec & 127)
            )
            plsc.store_scatter(slab, [loc], val, mask=m)

    def fire(slab, blk, sem):
        return pltpu.async_copy(slab, out_hbm.at[pl.ds(blk * BLK, BLK)], sem)

    def drain(slab, blk, sem):
        pltpu.make_async_copy(slab, out_hbm.at[pl.ds(blk * BLK, BLK)], sem).wait()

    for s in range(2):
        inject(slabs[s], s * NW + wid, ones16)
        fire(slabs[s], s * NW + wid, sems[s])

    @pl.loop(1, (SLOTS - 2) // 2, unroll=1)
    def _pair(p):
        for s in range(2):
            i = 2 * p + s
            prev = (i - 2) * NW + wid
            drain(slabs[s], prev, sems[s])
            inject(slabs[s], prev, zeros16)
            inject(slabs[s], i * NW + wid, ones16)
            fire(slabs[s], i * NW + wid, sems[s])

    i14 = SLOTS - 2
    drain(slabs[0], (i14 - 2) * NW + wid, sems[0])
    inject(slabs[0], (i14 - 2) * NW + wid, zeros16)
    inject(slabs[0], i14 * NW + wid, ones16)
    last0 = fire(slabs[0], i14 * NW + wid, sems[0])

    i15 = SLOTS - 1
    drain(slabs[1], (i15 - 2) * NW + wid, sems[1])
    inject(slabs[1], (i15 - 2) * NW + wid, zeros16)

    @pl.when(i15 * NW + wid < NBLK)
    def _():
        inject(slabs[1], i15 * NW + wid, ones16)
        fire(slabs[1], i15 * NW + wid, sems[1]).wait()

    last0.wait()


@jax.jit
def _one_hot_sc(t_flat, zeros_src):
    mesh = plsc.VectorSubcoreMesh(core_axis_name="c", subcore_axis_name="s")
    run = pl.kernel(
        _sc_body,
        out_type=jax.ShapeDtypeStruct((FLAT,), jnp.float32),
        mesh=mesh,
        scratch_types=[
            pltpu.VMEM((B * L,), jnp.int32),
            pltpu.VMEM((BLK,), jnp.float32),
            pltpu.VMEM((BLK,), jnp.float32),
            pltpu.SemaphoreType.DMA,
            pltpu.SemaphoreType.DMA,
            pltpu.SemaphoreType.DMA,
        ],
        compiler_params=pltpu.CompilerParams(needs_layout_passes=False),
        name="one_hot_sc",
    )
    flat = run(t_flat, zeros_src)
    return (
        flat.reshape(L, C // 8, B // 128, 8, 128)
        .transpose(2, 4, 1, 3, 0)
        .reshape(B, C, L)
    )


def kernel(t, ones):
    del ones
    t_flat = t.reshape(-1).astype(jnp.int32)
    zeros_src = jnp.zeros((BLK,), jnp.float32)
    return _one_hot_sc(t_flat, zeros_src)
